# Initial kernel scaffold; baseline (speedup 1.0000x reference)
import jax
import jax.numpy as jnp
from jax.experimental import pallas as pl


def kernel(patch_feats, patch_segment_ids, edge_index, edge_weights, W1, b1, W2, b2, W_out, b_out):
    raise NotImplementedError("write your pallas kernel here")


# speedup vs baseline: 3.5736x; 3.5736x over previous
import dataclasses

import jax
import jax.numpy as jnp
from jax import lax
from jax.experimental import pallas as pl
from jax.experimental.pallas import tpu as pltpu
from jax.experimental.pallas import tpu_sc as plsc

_sc_params = pltpu.CompilerParams()
if "needs_layout_passes" in pltpu.CompilerParams.__dataclass_fields__:
    _sc_params = dataclasses.replace(_sc_params, needs_layout_passes=False)

N_NODES = 10000
N_TOK = 500000
N_EDGES = 320000
D = 128
OUT_FEATS = 64

NC = 2
NS = 16
NW = NC * NS
TB = 128
NB_T = N_TOK // TB
TOK_TAIL = N_TOK - NB_T * TB
NB_E = N_EDGES // TB
BLK_T_CORE = NB_T // NC
BLK_E_CORE = NB_E // NC
T_ITERS = (BLK_T_CORE + NS - 1) // NS
E_ITERS = (BLK_E_CORE + NS - 1) // NS
RPT = 624
RREM = N_NODES - NS * RPT

HGRID = 80
HID = HGRID * 128
TOK_W = N_TOK // NW
TOK_WP = 15632
SENTINEL = 10100
EDG_W = N_EDGES // NW

EB_W = 79
EDG_WP = EB_W * TB
AGG_R = EDG_WP
RPT_E = AGG_R // NS

_mesh = plsc.VectorSubcoreMesh(core_axis_name="c", subcore_axis_name="s")

_f32 = jnp.float32
_i32 = jnp.int32


def _zero_buf(buf):
    rows, w = buf.shape

    @pl.loop(0, rows)
    def _(i):
        for k in range(w // 16):
            buf[i, pl.ds(k * 16, 16)] = jnp.zeros((16,), _f32)


def _stage_zero(acc, zbuf, s):
    @pl.loop(0, RPT // TB)
    def _(t):
        pltpu.sync_copy(zbuf, acc.at[pl.ds(s * RPT + t * TB, TB)])

    rem = RPT % TB
    pltpu.sync_copy(zbuf.at[pl.ds(0, rem)],
                    acc.at[pl.ds(s * RPT + (RPT // TB) * TB, rem)])

    @pl.when(s == NS - 1)
    def _():
        pltpu.sync_copy(zbuf.at[pl.ds(0, RREM)],
                        acc.at[pl.ds(NS * RPT, RREM)])


def _stage_out(acc, buf, hbm_dst, s):
    @pl.loop(0, RPT // TB)
    def _(t):
        o = s * RPT + t * TB
        pltpu.sync_copy(acc.at[pl.ds(o, TB)], buf)
        pltpu.sync_copy(buf, hbm_dst.at[pl.ds(o, TB)])

    rem = RPT % TB
    o2 = s * RPT + (RPT // TB) * TB
    pltpu.sync_copy(acc.at[pl.ds(o2, rem)], buf.at[pl.ds(0, rem)])
    pltpu.sync_copy(buf.at[pl.ds(0, rem)], hbm_dst.at[pl.ds(o2, rem)])

    @pl.when(s == NS - 1)
    def _():
        pltpu.sync_copy(acc.at[pl.ds(NS * RPT, RREM)], buf.at[pl.ds(0, RREM)])
        pltpu.sync_copy(buf.at[pl.ds(0, RREM)],
                        hbm_dst.at[pl.ds(NS * RPT, RREM)])


def _stats_body(feats, ids_main, ids_tail,
                sums_p,
                idx_v, idx32_v, rows_v, rows32_v,
                sum_acc):
    c = lax.axis_index("c")
    s = lax.axis_index("s")

    _zero_buf(rows_v)
    _stage_zero(sum_acc, rows_v, s)
    plsc.subcore_barrier()

    @pl.loop(0, T_ITERS)
    def _(j):
        bl = s + NS * j

        @pl.when(bl < BLK_T_CORE)
        def _():
            b = c * BLK_T_CORE + bl
            pltpu.sync_copy(ids_main.at[b], idx_v)
            pltpu.sync_copy(feats.at[pl.ds(b * TB, TB)], rows_v)
            pltpu.sync_copy(rows_v, sum_acc.at[idx_v.at[0]], add=True)

    @pl.when((c == 0) & (s == 0))
    def _():
        pltpu.sync_copy(ids_tail, idx32_v)
        pltpu.sync_copy(feats.at[pl.ds(NB_T * TB, TOK_TAIL)], rows32_v)
        pltpu.sync_copy(rows32_v, sum_acc.at[idx32_v.at[0]], add=True)

    plsc.subcore_barrier()

    _stage_out(sum_acc, rows_v, sums_p.at[c], s)


_sc_stats = pl.kernel(
    _stats_body,
    out_type=jax.ShapeDtypeStruct((NC, N_NODES, D), _f32),
    mesh=_mesh,
    scratch_types=[
        pltpu.VMEM((1, TB), _i32),
        pltpu.VMEM((1, TOK_TAIL), _i32),
        pltpu.VMEM((TB, D), _f32),
        pltpu.VMEM((TOK_TAIL, D), _f32),
        pltpu.VMEM_SHARED((N_NODES, D), _f32),
    ],
)


def _hist_update(hist, buf, n16):
    ones16 = jnp.ones((16,), _f32)

    @pl.loop(0, n16)
    def _(k):
        idv = buf[pl.ds(k * 16, 16)]
        rowv = jnp.right_shift(idv, 7)
        colv = jnp.bitwise_and(idv, 127)
        plsc.addupdate_scatter(hist, [rowv, colv], ones16)


def _deg_body(ids_pad, srcw, dstw, dep,
              cnt_p, odeg_p, ideg_p,
              idsb_v, eb_v, cnt_h, odeg_h, ideg_h):
    c = lax.axis_index("c")
    s = lax.axis_index("s")
    w = c * NS + s

    _zero_buf(cnt_h)
    _zero_buf(odeg_h)
    _zero_buf(ideg_h)

    pltpu.sync_copy(ids_pad.at[w], idsb_v)
    _hist_update(cnt_h, idsb_v, TOK_WP // 16)

    pltpu.sync_copy(srcw.at[w], eb_v)
    _hist_update(odeg_h, eb_v, EDG_W // 16)

    pltpu.sync_copy(dstw.at[w], eb_v)
    _hist_update(ideg_h, eb_v, EDG_W // 16)

    pltpu.sync_copy(cnt_h, cnt_p.at[w])
    pltpu.sync_copy(odeg_h, odeg_p.at[w])
    pltpu.sync_copy(ideg_h, ideg_p.at[w])


_sc_deg = pl.kernel(
    _deg_body,
    out_type=(
        jax.ShapeDtypeStruct((NW, HGRID, 128), _f32),
        jax.ShapeDtypeStruct((NW, HGRID, 128), _f32),
        jax.ShapeDtypeStruct((NW, HGRID, 128), _f32),
    ),
    mesh=_mesh,
    scratch_types=[
        pltpu.VMEM((TOK_WP,), _i32),
        pltpu.VMEM((EDG_W,), _i32),
        pltpu.VMEM((HGRID, 128), _f32),
        pltpu.VMEM((HGRID, 128), _f32),
        pltpu.VMEM((HGRID, 128), _f32),
    ],
    compiler_params=_sc_params,
)


def _stage_zero_e(acc, zbuf, s):
    @pl.loop(0, RPT_E // TB)
    def _(t):
        pltpu.sync_copy(zbuf, acc.at[pl.ds(s * RPT_E + t * TB, TB)])

    rem = RPT_E % TB
    pltpu.sync_copy(zbuf.at[pl.ds(0, rem)],
                    acc.at[pl.ds(s * RPT_E + (RPT_E // TB) * TB, rem)])


def _stage_out_e(acc, buf, hbm_dst, s):
    @pl.loop(0, RPT_E // TB)
    def _(t):
        o = s * RPT_E + t * TB
        pltpu.sync_copy(acc.at[pl.ds(o, TB)], buf)
        pltpu.sync_copy(buf, hbm_dst.at[pl.ds(o, TB)])

    rem = RPT_E % TB
    o2 = s * RPT_E + (RPT_E // TB) * TB
    pltpu.sync_copy(acc.at[pl.ds(o2, rem)], buf.at[pl.ds(0, rem)])
    pltpu.sync_copy(buf.at[pl.ds(0, rem)], hbm_dst.at[pl.ds(o2, rem)])


def _edge_body(xs, srcp, dstp, ew16p,
               agg_p,
               idxs_v, idxd_v, ew16_v, rows_v, agg_acc):
    c = lax.axis_index("c")
    s = lax.axis_index("s")
    w = c * NS + s

    _zero_buf(rows_v)
    _stage_zero_e(agg_acc, rows_v, s)
    plsc.subcore_barrier()

    @pl.loop(0, EB_W)
    def _(j):
        b = w * EB_W + j
        pltpu.sync_copy(srcp.at[b], idxs_v)
        pltpu.sync_copy(dstp.at[b], idxd_v)
        pltpu.sync_copy(ew16p.at[b], ew16_v)
        pltpu.sync_copy(xs.at[idxs_v.at[0]], rows_v)

        @pl.loop(0, TB)
        def _(i):
            wv = ew16_v[i, :]
            for k in range(D // 16):
                sl = pl.ds(k * 16, 16)
                rows_v[i, sl] = rows_v[i, sl] * wv

        pltpu.sync_copy(rows_v, agg_acc.at[idxd_v.at[0]], add=True)

    plsc.subcore_barrier()
    _stage_out_e(agg_acc, rows_v, agg_p.at[c], s)


_sc_edge = pl.kernel(
    _edge_body,
    out_type=jax.ShapeDtypeStruct((NC, AGG_R, D), _f32),
    mesh=_mesh,
    scratch_types=[
        pltpu.VMEM((1, TB), _i32),
        pltpu.VMEM((1, TB), _i32),
        pltpu.VMEM((TB, 16), _f32),
        pltpu.VMEM((TB, D), _f32),
        pltpu.VMEM_SHARED((AGG_R, D), _f32),
    ],
)


def _hist_total(p_ref):
    t = jnp.sum(p_ref[...], axis=0)
    t = t.reshape(HID)[:N_NODES]
    return t[:, None]


def _tc_prep_body(sums_p, cnt_p, odeg_p, xs1):
    sums = sums_p[0] + sums_p[1]
    cnt = jnp.maximum(_hist_total(cnt_p), 1.0)
    ro = sums / cnt
    od = jnp.maximum(_hist_total(odeg_p), 1.0)
    xs1[...] = ro * lax.rsqrt(od)


def _tc_prep(sums_p, cnt_p, odeg_p):
    return pl.pallas_call(
        _tc_prep_body,
        out_shape=jax.ShapeDtypeStruct((N_NODES, D), _f32),
    )(sums_p, cnt_p, odeg_p)


def _tc_conv_body(agg_p, ideg_p, odeg_p, W, b, xs2):
    a = agg_p[0, :N_NODES] + agg_p[1, :N_NODES]
    iv = jnp.maximum(_hist_total(ideg_p), 1.0)
    a = a * lax.rsqrt(iv)
    h = jnp.dot(a, W[...], preferred_element_type=_f32,
                precision=lax.Precision.HIGHEST) + b[...]
    h = jnp.maximum(h, 0.0)
    od = jnp.maximum(_hist_total(odeg_p), 1.0)
    xs2[...] = h * lax.rsqrt(od)


def _tc_conv(agg_p, ideg_p, odeg_p, W, b):
    return pl.pallas_call(
        _tc_conv_body,
        out_shape=jax.ShapeDtypeStruct((N_NODES, D), _f32),
    )(agg_p, ideg_p, odeg_p, W, b)


def _tc_head_body(agg_p, ideg_p, W2, b2, W_out, b_out, out):
    a = agg_p[0, :N_NODES] + agg_p[1, :N_NODES]
    iv = jnp.maximum(_hist_total(ideg_p), 1.0)
    a = a * lax.rsqrt(iv)
    h = jnp.dot(a, W2[...], preferred_element_type=_f32,
                precision=lax.Precision.HIGHEST) + b2[...]
    h = jnp.maximum(h, 0.0)
    g = jnp.mean(h, axis=0, keepdims=True)
    out[...] = jnp.dot(g, W_out[...], preferred_element_type=_f32,
                       precision=lax.Precision.HIGHEST) + b_out[...]


def _tc_head(agg_p, ideg_p, W2, b2, W_out, b_out):
    return pl.pallas_call(
        _tc_head_body,
        out_shape=jax.ShapeDtypeStruct((1, OUT_FEATS), _f32),
    )(agg_p, ideg_p, W2, b2, W_out, b_out)


def kernel(patch_feats, patch_segment_ids, edge_index, edge_weights,
           W1, b1, W2, b2, W_out, b_out):
    ids = patch_segment_ids.astype(_i32)
    src = edge_index[0].astype(_i32)
    dst = edge_index[1].astype(_i32)
    ew = edge_weights.astype(_f32)

    ids_main = ids[: NB_T * TB].reshape(NB_T, 1, TB)
    ids_tail = ids[NB_T * TB:].reshape(1, TOK_TAIL)

    ids_pad = jnp.full((NW, TOK_WP), SENTINEL, _i32)
    ids_pad = ids_pad.at[:, :TOK_W].set(ids.reshape(NW, TOK_W))
    srcw = src.reshape(NW, EDG_W)
    dstw = dst.reshape(NW, EDG_W)

    srcp = jnp.zeros((NW, EDG_WP), _i32)
    srcp = srcp.at[:, :EDG_W].set(srcw).reshape(NW * EB_W, 1, TB)
    dstp = jnp.full((NW, EDG_WP), N_NODES, _i32)
    dstp = dstp.at[:, :EDG_W].set(dstw).reshape(NW * EB_W, 1, TB)
    ewp = jnp.zeros((NW, EDG_WP), _f32)
    ewp = ewp.at[:, :EDG_W].set(ew.reshape(NW, EDG_W))
    ew16p = jnp.broadcast_to(ewp.reshape(NW * EDG_WP, 1), (NW * EDG_WP, 16))
    ew16p = ew16p.reshape(NW * EB_W, TB, 16)

    sums_p = _sc_stats(patch_feats, ids_main, ids_tail)
    cnt_p, odeg_p, ideg_p = _sc_deg(ids_pad, srcw, dstw, sums_p)

    xs1 = _tc_prep(sums_p, cnt_p, odeg_p)
    agg1 = _sc_edge(xs1, srcp, dstp, ew16p)
    xs2 = _tc_conv(agg1, ideg_p, odeg_p, W1, b1.reshape(1, D))
    agg2 = _sc_edge(xs2, srcp, dstp, ew16p)
    out = _tc_head(agg2, ideg_p, W2, b2.reshape(1, D),
                   W_out, b_out.reshape(1, OUT_FEATS))
    return out[0]


# speedup vs baseline: 3.8972x; 1.0906x over previous
import dataclasses

import jax
import jax.numpy as jnp
from jax import lax
from jax.experimental import pallas as pl
from jax.experimental.pallas import tpu as pltpu
from jax.experimental.pallas import tpu_sc as plsc

_sc_params = pltpu.CompilerParams()
if "needs_layout_passes" in pltpu.CompilerParams.__dataclass_fields__:
    _sc_params = dataclasses.replace(_sc_params, needs_layout_passes=False)

N_NODES = 10000
N_TOK = 500000
N_EDGES = 320000
D = 128
OUT_FEATS = 64

NC = 2
NS = 16
NW = NC * NS
TB = 128
NB_T = N_TOK // TB
TOK_TAIL = N_TOK - NB_T * TB
NB_E = N_EDGES // TB
BLK_T_CORE = NB_T // NC
BLK_E_CORE = NB_E // NC
T_ITERS = (BLK_T_CORE + NS - 1) // NS
E_ITERS = (BLK_E_CORE + NS - 1) // NS
RPT = 624
RREM = N_NODES - NS * RPT

HGRID = 80
HID = HGRID * 128
TOK_W = N_TOK // NW
TOK_WP = 15632
SENTINEL = 10100
EDG_W = N_EDGES // NW

EB_W = 79
EDG_WP = EB_W * TB
AGG_R = EDG_WP
RPT_E = AGG_R // NS

_mesh = plsc.VectorSubcoreMesh(core_axis_name="c", subcore_axis_name="s")

_f32 = jnp.float32
_i32 = jnp.int32


def _zero_buf(buf):
    rows, w = buf.shape

    @pl.loop(0, rows)
    def _(i):
        for k in range(w // 16):
            buf[i, pl.ds(k * 16, 16)] = jnp.zeros((16,), _f32)


def _stage_zero(acc, zbuf, s):
    @pl.loop(0, RPT // TB)
    def _(t):
        pltpu.sync_copy(zbuf, acc.at[pl.ds(s * RPT + t * TB, TB)])

    rem = RPT % TB
    pltpu.sync_copy(zbuf.at[pl.ds(0, rem)],
                    acc.at[pl.ds(s * RPT + (RPT // TB) * TB, rem)])

    @pl.when(s == NS - 1)
    def _():
        pltpu.sync_copy(zbuf.at[pl.ds(0, RREM)],
                        acc.at[pl.ds(NS * RPT, RREM)])


def _stage_out(acc, buf, hbm_dst, s):
    @pl.loop(0, RPT // TB)
    def _(t):
        o = s * RPT + t * TB
        pltpu.sync_copy(acc.at[pl.ds(o, TB)], buf)
        pltpu.sync_copy(buf, hbm_dst.at[pl.ds(o, TB)])

    rem = RPT % TB
    o2 = s * RPT + (RPT // TB) * TB
    pltpu.sync_copy(acc.at[pl.ds(o2, rem)], buf.at[pl.ds(0, rem)])
    pltpu.sync_copy(buf.at[pl.ds(0, rem)], hbm_dst.at[pl.ds(o2, rem)])

    @pl.when(s == NS - 1)
    def _():
        pltpu.sync_copy(acc.at[pl.ds(NS * RPT, RREM)], buf.at[pl.ds(0, RREM)])
        pltpu.sync_copy(buf.at[pl.ds(0, RREM)],
                        hbm_dst.at[pl.ds(NS * RPT, RREM)])


TBW = 3904 // NW


def _stats_body(feats, ids_main, ids_tail,
                sums_p,
                idx_a, idx_b, idx32_v, rows_a, rows_b, rows32_v,
                sg_a, sg_b,
                sum_acc):
    c = lax.axis_index("c")
    s = lax.axis_index("s")
    w = c * NS + s
    base = w * TBW
    banks = [(idx_a, rows_a, sg_a), (idx_b, rows_b, sg_b)]

    _zero_buf(rows_a)
    _stage_zero(sum_acc, rows_a, s)
    plsc.subcore_barrier()

    pltpu.sync_copy(ids_main.at[base], idx_a)
    pltpu.async_copy(feats.at[pl.ds(base * TB, TB)], rows_a, sg_a)

    @pl.loop(0, TBW // 2)
    def _(jo):
        for p in (0, 1):
            j = 2 * jo + p
            idx, rows, sg = banks[p]
            oidx, orows, osg = banks[1 - p]

            nxt = base + jnp.minimum(j + 1, TBW - 1)
            pltpu.sync_copy(ids_main.at[nxt], oidx)
            pltpu.async_copy(feats.at[pl.ds(nxt * TB, TB)], orows, osg)

            pltpu.make_async_copy(feats.at[pl.ds(base * TB, TB)], rows,
                                  sg).wait()
            pltpu.sync_copy(rows, sum_acc.at[idx.at[0]], add=True)

    pltpu.make_async_copy(feats.at[pl.ds(base * TB, TB)], rows_a, sg_a).wait()

    @pl.when((c == 0) & (s == 0))
    def _():
        for bb in (NW * TBW, NW * TBW + 1):
            pltpu.sync_copy(ids_main.at[bb], idx_a)
            pltpu.sync_copy(feats.at[pl.ds(bb * TB, TB)], rows_a)
            pltpu.sync_copy(rows_a, sum_acc.at[idx_a.at[0]], add=True)
        pltpu.sync_copy(ids_tail, idx32_v)
        pltpu.sync_copy(feats.at[pl.ds(NB_T * TB, TOK_TAIL)], rows32_v)
        pltpu.sync_copy(rows32_v, sum_acc.at[idx32_v.at[0]], add=True)

    plsc.subcore_barrier()

    _stage_out(sum_acc, rows_a, sums_p.at[c], s)


_sc_stats = pl.kernel(
    _stats_body,
    out_type=jax.ShapeDtypeStruct((NC, N_NODES, D), _f32),
    mesh=_mesh,
    scratch_types=[
        pltpu.VMEM((1, TB), _i32),
        pltpu.VMEM((1, TB), _i32),
        pltpu.VMEM((1, TOK_TAIL), _i32),
        pltpu.VMEM((TB, D), _f32),
        pltpu.VMEM((TB, D), _f32),
        pltpu.VMEM((TOK_TAIL, D), _f32),
        pltpu.SemaphoreType.DMA,
        pltpu.SemaphoreType.DMA,
        pltpu.VMEM_SHARED((N_NODES, D), _f32),
    ],
)


def _hist_update(hist, buf, n16):
    ones16 = jnp.ones((16,), _f32)

    @pl.loop(0, n16)
    def _(k):
        idv = buf[pl.ds(k * 16, 16)]
        rowv = jnp.right_shift(idv, 7)
        colv = jnp.bitwise_and(idv, 127)
        plsc.addupdate_scatter(hist, [rowv, colv], ones16)


def _deg_body(ids_pad, srcw, dstw, dep,
              cnt_p, odeg_p, ideg_p,
              idsb_v, eb_v, cnt_h, odeg_h, ideg_h):
    c = lax.axis_index("c")
    s = lax.axis_index("s")
    w = c * NS + s

    _zero_buf(cnt_h)
    _zero_buf(odeg_h)
    _zero_buf(ideg_h)

    pltpu.sync_copy(ids_pad.at[w], idsb_v)
    _hist_update(cnt_h, idsb_v, TOK_WP // 16)

    pltpu.sync_copy(srcw.at[w], eb_v)
    _hist_update(odeg_h, eb_v, EDG_W // 16)

    pltpu.sync_copy(dstw.at[w], eb_v)
    _hist_update(ideg_h, eb_v, EDG_W // 16)

    pltpu.sync_copy(cnt_h, cnt_p.at[w])
    pltpu.sync_copy(odeg_h, odeg_p.at[w])
    pltpu.sync_copy(ideg_h, ideg_p.at[w])


_sc_deg = pl.kernel(
    _deg_body,
    out_type=(
        jax.ShapeDtypeStruct((NW, HGRID, 128), _f32),
        jax.ShapeDtypeStruct((NW, HGRID, 128), _f32),
        jax.ShapeDtypeStruct((NW, HGRID, 128), _f32),
    ),
    mesh=_mesh,
    scratch_types=[
        pltpu.VMEM((TOK_WP,), _i32),
        pltpu.VMEM((EDG_W,), _i32),
        pltpu.VMEM((HGRID, 128), _f32),
        pltpu.VMEM((HGRID, 128), _f32),
        pltpu.VMEM((HGRID, 128), _f32),
    ],
    compiler_params=_sc_params,
)


def _stage_zero_e(acc, zbuf, s):
    @pl.loop(0, RPT_E // TB)
    def _(t):
        pltpu.sync_copy(zbuf, acc.at[pl.ds(s * RPT_E + t * TB, TB)])

    rem = RPT_E % TB
    pltpu.sync_copy(zbuf.at[pl.ds(0, rem)],
                    acc.at[pl.ds(s * RPT_E + (RPT_E // TB) * TB, rem)])


def _stage_out_e(acc, buf, hbm_dst, s):
    @pl.loop(0, RPT_E // TB)
    def _(t):
        o = s * RPT_E + t * TB
        pltpu.sync_copy(acc.at[pl.ds(o, TB)], buf)
        pltpu.sync_copy(buf, hbm_dst.at[pl.ds(o, TB)])

    rem = RPT_E % TB
    o2 = s * RPT_E + (RPT_E // TB) * TB
    pltpu.sync_copy(acc.at[pl.ds(o2, rem)], buf.at[pl.ds(0, rem)])
    pltpu.sync_copy(buf.at[pl.ds(0, rem)], hbm_dst.at[pl.ds(o2, rem)])


def _edge_body(xs, srcp, dstp, ew16p,
               agg_p,
               idxs_v, idxd_v, ew16_v, rows_v, agg_acc):
    c = lax.axis_index("c")
    s = lax.axis_index("s")
    w = c * NS + s

    _zero_buf(rows_v)
    _stage_zero_e(agg_acc, rows_v, s)
    plsc.subcore_barrier()

    @pl.loop(0, EB_W)
    def _(j):
        b = w * EB_W + j
        pltpu.sync_copy(srcp.at[b], idxs_v)
        pltpu.sync_copy(dstp.at[b], idxd_v)
        pltpu.sync_copy(ew16p.at[b], ew16_v)
        pltpu.sync_copy(xs.at[idxs_v.at[0]], rows_v)

        @pl.loop(0, TB)
        def _(i):
            wv = ew16_v[i, :]
            for k in range(D // 16):
                sl = pl.ds(k * 16, 16)
                rows_v[i, sl] = rows_v[i, sl] * wv

        pltpu.sync_copy(rows_v, agg_acc.at[idxd_v.at[0]], add=True)

    plsc.subcore_barrier()
    _stage_out_e(agg_acc, rows_v, agg_p.at[c], s)


_sc_edge = pl.kernel(
    _edge_body,
    out_type=jax.ShapeDtypeStruct((NC, AGG_R, D), _f32),
    mesh=_mesh,
    scratch_types=[
        pltpu.VMEM((1, TB), _i32),
        pltpu.VMEM((1, TB), _i32),
        pltpu.VMEM((TB, 16), _f32),
        pltpu.VMEM((TB, D), _f32),
        pltpu.VMEM_SHARED((AGG_R, D), _f32),
    ],
)


def _hist_total(p_ref):
    t = jnp.sum(p_ref[...], axis=0)
    t = t.reshape(HID)[:N_NODES]
    return t[:, None]


def _tc_prep_body(sums_p, cnt_p, odeg_p, xs1):
    sums = sums_p[0] + sums_p[1]
    cnt = jnp.maximum(_hist_total(cnt_p), 1.0)
    ro = sums / cnt
    od = jnp.maximum(_hist_total(odeg_p), 1.0)
    xs1[...] = ro * lax.rsqrt(od)


def _tc_prep(sums_p, cnt_p, odeg_p):
    return pl.pallas_call(
        _tc_prep_body,
        out_shape=jax.ShapeDtypeStruct((N_NODES, D), _f32),
    )(sums_p, cnt_p, odeg_p)


def _tc_conv_body(agg_p, ideg_p, odeg_p, W, b, xs2):
    a = agg_p[0, :N_NODES] + agg_p[1, :N_NODES]
    iv = jnp.maximum(_hist_total(ideg_p), 1.0)
    a = a * lax.rsqrt(iv)
    h = jnp.dot(a, W[...], preferred_element_type=_f32,
                precision=lax.Precision.HIGHEST) + b[...]
    h = jnp.maximum(h, 0.0)
    od = jnp.maximum(_hist_total(odeg_p), 1.0)
    xs2[...] = h * lax.rsqrt(od)


def _tc_conv(agg_p, ideg_p, odeg_p, W, b):
    return pl.pallas_call(
        _tc_conv_body,
        out_shape=jax.ShapeDtypeStruct((N_NODES, D), _f32),
    )(agg_p, ideg_p, odeg_p, W, b)


def _tc_head_body(agg_p, ideg_p, W2, b2, W_out, b_out, out):
    a = agg_p[0, :N_NODES] + agg_p[1, :N_NODES]
    iv = jnp.maximum(_hist_total(ideg_p), 1.0)
    a = a * lax.rsqrt(iv)
    h = jnp.dot(a, W2[...], preferred_element_type=_f32,
                precision=lax.Precision.HIGHEST) + b2[...]
    h = jnp.maximum(h, 0.0)
    g = jnp.mean(h, axis=0, keepdims=True)
    out[...] = jnp.dot(g, W_out[...], preferred_element_type=_f32,
                       precision=lax.Precision.HIGHEST) + b_out[...]


def _tc_head(agg_p, ideg_p, W2, b2, W_out, b_out):
    return pl.pallas_call(
        _tc_head_body,
        out_shape=jax.ShapeDtypeStruct((1, OUT_FEATS), _f32),
    )(agg_p, ideg_p, W2, b2, W_out, b_out)


def kernel(patch_feats, patch_segment_ids, edge_index, edge_weights,
           W1, b1, W2, b2, W_out, b_out):
    ids = patch_segment_ids.astype(_i32)
    src = edge_index[0].astype(_i32)
    dst = edge_index[1].astype(_i32)
    ew = edge_weights.astype(_f32)

    ids_main = ids[: NB_T * TB].reshape(NB_T, 1, TB)
    ids_tail = ids[NB_T * TB:].reshape(1, TOK_TAIL)

    ids_pad = jnp.full((NW, TOK_WP), SENTINEL, _i32)
    ids_pad = ids_pad.at[:, :TOK_W].set(ids.reshape(NW, TOK_W))
    srcw = src.reshape(NW, EDG_W)
    dstw = dst.reshape(NW, EDG_W)

    srcp = jnp.zeros((NW, EDG_WP), _i32)
    srcp = srcp.at[:, :EDG_W].set(srcw).reshape(NW * EB_W, 1, TB)
    dstp = jnp.full((NW, EDG_WP), N_NODES, _i32)
    dstp = dstp.at[:, :EDG_W].set(dstw).reshape(NW * EB_W, 1, TB)
    ewp = jnp.zeros((NW, EDG_WP), _f32)
    ewp = ewp.at[:, :EDG_W].set(ew.reshape(NW, EDG_W))
    ew16p = jnp.broadcast_to(ewp.reshape(NW * EDG_WP, 1), (NW * EDG_WP, 16))
    ew16p = ew16p.reshape(NW * EB_W, TB, 16)

    sums_p = _sc_stats(patch_feats, ids_main, ids_tail)
    cnt_p, odeg_p, ideg_p = _sc_deg(ids_pad, srcw, dstw, sums_p)

    xs1 = _tc_prep(sums_p, cnt_p, odeg_p)
    agg1 = _sc_edge(xs1, srcp, dstp, ew16p)
    xs2 = _tc_conv(agg1, ideg_p, odeg_p, W1, b1.reshape(1, D))
    agg2 = _sc_edge(xs2, srcp, dstp, ew16p)
    out = _tc_head(agg2, ideg_p, W2, b2.reshape(1, D),
                   W_out, b_out.reshape(1, OUT_FEATS))
    return out[0]


# speedup vs baseline: 3.8997x; 1.0006x over previous
import dataclasses

import jax
import jax.numpy as jnp
from jax import lax
from jax.experimental import pallas as pl
from jax.experimental.pallas import tpu as pltpu
from jax.experimental.pallas import tpu_sc as plsc

_sc_params = pltpu.CompilerParams()
if "needs_layout_passes" in pltpu.CompilerParams.__dataclass_fields__:
    _sc_params = dataclasses.replace(_sc_params, needs_layout_passes=False)

N_NODES = 10000
N_TOK = 500000
N_EDGES = 320000
D = 128
OUT_FEATS = 64

NC = 2
NS = 16
NW = NC * NS
TB = 128
NB_T = N_TOK // TB
TOK_TAIL = N_TOK - NB_T * TB
NB_E = N_EDGES // TB
BLK_T_CORE = NB_T // NC
BLK_E_CORE = NB_E // NC
T_ITERS = (BLK_T_CORE + NS - 1) // NS
E_ITERS = (BLK_E_CORE + NS - 1) // NS
RPT = 624
RREM = N_NODES - NS * RPT

HGRID = 80
HID = HGRID * 128
TOK_W = N_TOK // NW
TOK_WP = 15632
SENTINEL = 10100
EDG_W = N_EDGES // NW

EB_W = 79
EDG_WP = EB_W * TB
AGG_R = EDG_WP
RPT_E = AGG_R // NS

_mesh = plsc.VectorSubcoreMesh(core_axis_name="c", subcore_axis_name="s")

_f32 = jnp.float32
_i32 = jnp.int32


def _zero_buf(buf):
    rows, w = buf.shape

    @pl.loop(0, rows)
    def _(i):
        for k in range(w // 16):
            buf[i, pl.ds(k * 16, 16)] = jnp.zeros((16,), _f32)


def _stage_zero(acc, zbuf, s):
    @pl.loop(0, RPT // TB)
    def _(t):
        pltpu.sync_copy(zbuf, acc.at[pl.ds(s * RPT + t * TB, TB)])

    rem = RPT % TB
    pltpu.sync_copy(zbuf.at[pl.ds(0, rem)],
                    acc.at[pl.ds(s * RPT + (RPT // TB) * TB, rem)])

    @pl.when(s == NS - 1)
    def _():
        pltpu.sync_copy(zbuf.at[pl.ds(0, RREM)],
                        acc.at[pl.ds(NS * RPT, RREM)])


def _stage_out(acc, buf, hbm_dst, s):
    @pl.loop(0, RPT // TB)
    def _(t):
        o = s * RPT + t * TB
        pltpu.sync_copy(acc.at[pl.ds(o, TB)], buf)
        pltpu.sync_copy(buf, hbm_dst.at[pl.ds(o, TB)])

    rem = RPT % TB
    o2 = s * RPT + (RPT // TB) * TB
    pltpu.sync_copy(acc.at[pl.ds(o2, rem)], buf.at[pl.ds(0, rem)])
    pltpu.sync_copy(buf.at[pl.ds(0, rem)], hbm_dst.at[pl.ds(o2, rem)])

    @pl.when(s == NS - 1)
    def _():
        pltpu.sync_copy(acc.at[pl.ds(NS * RPT, RREM)], buf.at[pl.ds(0, RREM)])
        pltpu.sync_copy(buf.at[pl.ds(0, RREM)],
                        hbm_dst.at[pl.ds(NS * RPT, RREM)])


TBW = 3904 // NW


def _stats_body(feats, ids_main, ids_tail,
                sums_p,
                idx_a, idx_b, idx32_v, rows_a, rows_b, rows32_v,
                sg_a, sg_b,
                sum_acc):
    c = lax.axis_index("c")
    s = lax.axis_index("s")
    w = c * NS + s
    base = w * TBW
    banks = [(idx_a, rows_a, sg_a), (idx_b, rows_b, sg_b)]

    _zero_buf(rows_a)
    _stage_zero(sum_acc, rows_a, s)
    plsc.subcore_barrier()

    pltpu.sync_copy(ids_main.at[base], idx_a)
    pltpu.async_copy(feats.at[pl.ds(base * TB, TB)], rows_a, sg_a)

    @pl.loop(0, TBW // 2)
    def _(jo):
        for p in (0, 1):
            j = 2 * jo + p
            idx, rows, sg = banks[p]
            oidx, orows, osg = banks[1 - p]

            nxt = base + jnp.minimum(j + 1, TBW - 1)
            pltpu.sync_copy(ids_main.at[nxt], oidx)
            pltpu.async_copy(feats.at[pl.ds(nxt * TB, TB)], orows, osg)

            pltpu.make_async_copy(feats.at[pl.ds(base * TB, TB)], rows,
                                  sg).wait()
            pltpu.sync_copy(rows, sum_acc.at[idx.at[0]], add=True)

    pltpu.make_async_copy(feats.at[pl.ds(base * TB, TB)], rows_a, sg_a).wait()

    @pl.when((c == 0) & (s == 0))
    def _():
        for bb in (NW * TBW, NW * TBW + 1):
            pltpu.sync_copy(ids_main.at[bb], idx_a)
            pltpu.sync_copy(feats.at[pl.ds(bb * TB, TB)], rows_a)
            pltpu.sync_copy(rows_a, sum_acc.at[idx_a.at[0]], add=True)
        pltpu.sync_copy(ids_tail, idx32_v)
        pltpu.sync_copy(feats.at[pl.ds(NB_T * TB, TOK_TAIL)], rows32_v)
        pltpu.sync_copy(rows32_v, sum_acc.at[idx32_v.at[0]], add=True)

    plsc.subcore_barrier()

    _stage_out(sum_acc, rows_a, sums_p.at[c], s)


_sc_stats = pl.kernel(
    _stats_body,
    out_type=jax.ShapeDtypeStruct((NC, N_NODES, D), _f32),
    mesh=_mesh,
    scratch_types=[
        pltpu.VMEM((1, TB), _i32),
        pltpu.VMEM((1, TB), _i32),
        pltpu.VMEM((1, TOK_TAIL), _i32),
        pltpu.VMEM((TB, D), _f32),
        pltpu.VMEM((TB, D), _f32),
        pltpu.VMEM((TOK_TAIL, D), _f32),
        pltpu.SemaphoreType.DMA,
        pltpu.SemaphoreType.DMA,
        pltpu.VMEM_SHARED((N_NODES, D), _f32),
    ],
)


def _hist_update(hist, buf, n16):
    ones16 = jnp.ones((16,), _f32)

    @pl.loop(0, n16)
    def _(k):
        idv = buf[pl.ds(k * 16, 16)]
        rowv = jnp.right_shift(idv, 7)
        colv = jnp.bitwise_and(idv, 127)
        plsc.addupdate_scatter(hist, [rowv, colv], ones16)


def _deg_body(ids_pad, srcw, dstw, dep,
              cnt_p, odeg_p, ideg_p,
              idsb_v, eb_v, cnt_h, odeg_h, ideg_h):
    c = lax.axis_index("c")
    s = lax.axis_index("s")
    w = c * NS + s

    _zero_buf(cnt_h)
    _zero_buf(odeg_h)
    _zero_buf(ideg_h)

    pltpu.sync_copy(ids_pad.at[w], idsb_v)
    _hist_update(cnt_h, idsb_v, TOK_WP // 16)

    pltpu.sync_copy(srcw.at[w], eb_v)
    _hist_update(odeg_h, eb_v, EDG_W // 16)

    pltpu.sync_copy(dstw.at[w], eb_v)
    _hist_update(ideg_h, eb_v, EDG_W // 16)

    pltpu.sync_copy(cnt_h, cnt_p.at[w])
    pltpu.sync_copy(odeg_h, odeg_p.at[w])
    pltpu.sync_copy(ideg_h, ideg_p.at[w])


_sc_deg = pl.kernel(
    _deg_body,
    out_type=(
        jax.ShapeDtypeStruct((NW, HGRID, 128), _f32),
        jax.ShapeDtypeStruct((NW, HGRID, 128), _f32),
        jax.ShapeDtypeStruct((NW, HGRID, 128), _f32),
    ),
    mesh=_mesh,
    scratch_types=[
        pltpu.VMEM((TOK_WP,), _i32),
        pltpu.VMEM((EDG_W,), _i32),
        pltpu.VMEM((HGRID, 128), _f32),
        pltpu.VMEM((HGRID, 128), _f32),
        pltpu.VMEM((HGRID, 128), _f32),
    ],
    compiler_params=_sc_params,
)


def _stage_zero_e(acc, zbuf, s):
    @pl.loop(0, RPT_E // TB)
    def _(t):
        pltpu.sync_copy(zbuf, acc.at[pl.ds(s * RPT_E + t * TB, TB)])

    rem = RPT_E % TB
    pltpu.sync_copy(zbuf.at[pl.ds(0, rem)],
                    acc.at[pl.ds(s * RPT_E + (RPT_E // TB) * TB, rem)])


def _stage_out_e(acc, buf, hbm_dst, s):
    @pl.loop(0, RPT_E // TB)
    def _(t):
        o = s * RPT_E + t * TB
        pltpu.sync_copy(acc.at[pl.ds(o, TB)], buf)
        pltpu.sync_copy(buf, hbm_dst.at[pl.ds(o, TB)])

    rem = RPT_E % TB
    o2 = s * RPT_E + (RPT_E // TB) * TB
    pltpu.sync_copy(acc.at[pl.ds(o2, rem)], buf.at[pl.ds(0, rem)])
    pltpu.sync_copy(buf.at[pl.ds(0, rem)], hbm_dst.at[pl.ds(o2, rem)])


def _edge_body(xs, srcp, dstp, ew16p,
               agg_p,
               idxs_v, idxd_v, ew16_v, rows_v, agg_acc):
    c = lax.axis_index("c")
    s = lax.axis_index("s")
    w = c * NS + s

    _zero_buf(rows_v)
    _stage_zero_e(agg_acc, rows_v, s)
    plsc.subcore_barrier()

    @pl.loop(0, EB_W)
    def _(j):
        b = w * EB_W + j
        pltpu.sync_copy(srcp.at[b], idxs_v)
        pltpu.sync_copy(dstp.at[b], idxd_v)
        pltpu.sync_copy(ew16p.at[b], ew16_v)
        pltpu.sync_copy(xs.at[idxs_v.at[0]], rows_v)

        @pl.loop(0, TB // 4)
        def _(ih):
            for r in range(4):
                i = 4 * ih + r
                wv = ew16_v[i, :]
                for k in range(D // 16):
                    sl = pl.ds(k * 16, 16)
                    rows_v[i, sl] = rows_v[i, sl] * wv

        pltpu.sync_copy(rows_v, agg_acc.at[idxd_v.at[0]], add=True)

    plsc.subcore_barrier()
    _stage_out_e(agg_acc, rows_v, agg_p.at[c], s)


_sc_edge = pl.kernel(
    _edge_body,
    out_type=jax.ShapeDtypeStruct((NC, AGG_R, D), _f32),
    mesh=_mesh,
    scratch_types=[
        pltpu.VMEM((1, TB), _i32),
        pltpu.VMEM((1, TB), _i32),
        pltpu.VMEM((TB, 16), _f32),
        pltpu.VMEM((TB, D), _f32),
        pltpu.VMEM_SHARED((AGG_R, D), _f32),
    ],
)


def _hist_total(p_ref):
    t = jnp.sum(p_ref[...], axis=0)
    t = t.reshape(HID)[:N_NODES]
    return t[:, None]


def _tc_prep_body(sums_p, cnt_p, odeg_p, xs1):
    sums = sums_p[0] + sums_p[1]
    cnt = jnp.maximum(_hist_total(cnt_p), 1.0)
    ro = sums / cnt
    od = jnp.maximum(_hist_total(odeg_p), 1.0)
    xs1[...] = ro * lax.rsqrt(od)


def _tc_prep(sums_p, cnt_p, odeg_p):
    return pl.pallas_call(
        _tc_prep_body,
        out_shape=jax.ShapeDtypeStruct((N_NODES, D), _f32),
    )(sums_p, cnt_p, odeg_p)


def _tc_conv_body(agg_p, ideg_p, odeg_p, W, b, xs2):
    a = agg_p[0, :N_NODES] + agg_p[1, :N_NODES]
    iv = jnp.maximum(_hist_total(ideg_p), 1.0)
    a = a * lax.rsqrt(iv)
    h = jnp.dot(a, W[...], preferred_element_type=_f32,
                precision=lax.Precision.HIGHEST) + b[...]
    h = jnp.maximum(h, 0.0)
    od = jnp.maximum(_hist_total(odeg_p), 1.0)
    xs2[...] = h * lax.rsqrt(od)


def _tc_conv(agg_p, ideg_p, odeg_p, W, b):
    return pl.pallas_call(
        _tc_conv_body,
        out_shape=jax.ShapeDtypeStruct((N_NODES, D), _f32),
    )(agg_p, ideg_p, odeg_p, W, b)


def _tc_head_body(agg_p, ideg_p, W2, b2, W_out, b_out, out):
    a = agg_p[0, :N_NODES] + agg_p[1, :N_NODES]
    iv = jnp.maximum(_hist_total(ideg_p), 1.0)
    a = a * lax.rsqrt(iv)
    h = jnp.dot(a, W2[...], preferred_element_type=_f32,
                precision=lax.Precision.HIGHEST) + b2[...]
    h = jnp.maximum(h, 0.0)
    g = jnp.mean(h, axis=0, keepdims=True)
    out[...] = jnp.dot(g, W_out[...], preferred_element_type=_f32,
                       precision=lax.Precision.HIGHEST) + b_out[...]


def _tc_head(agg_p, ideg_p, W2, b2, W_out, b_out):
    return pl.pallas_call(
        _tc_head_body,
        out_shape=jax.ShapeDtypeStruct((1, OUT_FEATS), _f32),
    )(agg_p, ideg_p, W2, b2, W_out, b_out)


def kernel(patch_feats, patch_segment_ids, edge_index, edge_weights,
           W1, b1, W2, b2, W_out, b_out):
    ids = patch_segment_ids.astype(_i32)
    src = edge_index[0].astype(_i32)
    dst = edge_index[1].astype(_i32)
    ew = edge_weights.astype(_f32)

    ids_main = ids[: NB_T * TB].reshape(NB_T, 1, TB)
    ids_tail = ids[NB_T * TB:].reshape(1, TOK_TAIL)

    ids_pad = jnp.full((NW, TOK_WP), SENTINEL, _i32)
    ids_pad = ids_pad.at[:, :TOK_W].set(ids.reshape(NW, TOK_W))
    srcw = src.reshape(NW, EDG_W)
    dstw = dst.reshape(NW, EDG_W)

    srcp = jnp.zeros((NW, EDG_WP), _i32)
    srcp = srcp.at[:, :EDG_W].set(srcw).reshape(NW * EB_W, 1, TB)
    dstp = jnp.full((NW, EDG_WP), N_NODES, _i32)
    dstp = dstp.at[:, :EDG_W].set(dstw).reshape(NW * EB_W, 1, TB)
    ewp = jnp.zeros((NW, EDG_WP), _f32)
    ewp = ewp.at[:, :EDG_W].set(ew.reshape(NW, EDG_W))
    ew16p = jnp.broadcast_to(ewp.reshape(NW * EDG_WP, 1), (NW * EDG_WP, 16))
    ew16p = ew16p.reshape(NW * EB_W, TB, 16)

    sums_p = _sc_stats(patch_feats, ids_main, ids_tail)
    cnt_p, odeg_p, ideg_p = _sc_deg(ids_pad, srcw, dstw, sums_p)

    xs1 = _tc_prep(sums_p, cnt_p, odeg_p)
    agg1 = _sc_edge(xs1, srcp, dstp, ew16p)
    xs2 = _tc_conv(agg1, ideg_p, odeg_p, W1, b1.reshape(1, D))
    agg2 = _sc_edge(xs2, srcp, dstp, ew16p)
    out = _tc_head(agg2, ideg_p, W2, b2.reshape(1, D),
                   W_out, b_out.reshape(1, OUT_FEATS))
    return out[0]
